# baseline (speedup 1.0000x reference)
import functools

import jax
import jax.numpy as jnp
import numpy as np
from jax import lax
from jax.experimental import pallas as pl
from jax.experimental.pallas import tpu as pltpu
from jax.experimental.pallas import tpu_sc as plsc

B, N, D = 2, 2048, 256
K = 64
H = 8
DH = D // H
SCALE = 1.0 / float(np.sqrt(DH))
RT = 256
P = 32
PAIRS = P * K
NT1 = N // RT
NT3 = N // P

_NC, _NS = 2, 16
_NW = _NC * _NS
_TOTAL_PAIRS = B * N * K
_IDX_PER_W = _TOTAL_PAIRS // _NW
_CH = 128


def _k1_body(fb_ref, pts8_ref, ptsT8_ref, wqT_ref, wkT_ref, wvT_ref,
             bq_ref, bk_ref, bv_ref, w3T_ref, b3_ref,
             qf_ref, kv_ref, akT_ref, avT_ref, ckv_ref,
             idx_ref, nd_ref):
    b = pl.program_id(0)
    fb = fb_ref[0]
    wkT = wkT_ref[...]
    wvT = wvT_ref[...]
    qf_ref[0] = jnp.dot(fb, wqT_ref[...], preferred_element_type=jnp.float32) + bq_ref[...]
    kf = jnp.dot(fb, wkT, preferred_element_type=jnp.float32) + bk_ref[...]
    vf = jnp.dot(fb, wvT, preferred_element_type=jnp.float32) + bv_ref[...]
    kv_ref[0] = jnp.concatenate([kf, vf], axis=1).astype(jnp.bfloat16)
    w3T = w3T_ref[...]
    akT_ref[...] = jnp.dot(w3T, wkT, preferred_element_type=jnp.float32).astype(jnp.bfloat16)
    avT_ref[...] = jnp.dot(w3T, wvT, preferred_element_type=jnp.float32).astype(jnp.bfloat16)
    b3 = b3_ref[...]
    ck = jnp.dot(b3, wkT, preferred_element_type=jnp.float32)
    cv = jnp.dot(b3, wvT, preferred_element_type=jnp.float32)
    ckv_ref[...] = jnp.concatenate([ck, cv], axis=1)

    pts = pts8_ref[0]
    ptsT = ptsT8_ref[0]
    sq_row = jnp.sum(pts * pts, axis=1, keepdims=True)
    sq_all = jnp.sum(ptsT * ptsT, axis=0, keepdims=True)
    g = jnp.dot(pts, ptsT, preferred_element_type=jnp.float32)
    d2 = jnp.maximum(sq_row + sq_all - 2.0 * g, 1e-12)

    iota = lax.broadcasted_iota(jnp.int32, (RT, N), 1)
    slot = lax.broadcasted_iota(jnp.int32, (1, K), 1)

    def body(t, carry):
        work, acc_i, acc_n = carry
        m = jnp.min(work, axis=1, keepdims=True)
        sel = jnp.where(work == m, iota, jnp.int32(1 << 30))
        j = jnp.min(sel, axis=1, keepdims=True)
        onehot = slot == t
        acc_i = acc_i + jnp.where(onehot, j, 0)
        acc_n = acc_n + jnp.where(onehot, m, 0.0)
        work = jnp.where(iota == j, jnp.float32(np.inf), work)
        return work, acc_i, acc_n

    _, acc_i, acc_n = lax.fori_loop(
        0, K, body,
        (d2, jnp.zeros((RT, K), jnp.int32), jnp.zeros((RT, K), jnp.float32)))
    idx_ref[0] = acc_i + b * N
    nd_ref[0] = jnp.sqrt(acc_n)


def _stage1(fb, pts8, ptsT8, wqT, wkT, wvT, bq, bk, bv, w3T, b3):
    f32 = jnp.float32
    return pl.pallas_call(
        _k1_body,
        grid=(B, NT1),
        in_specs=[
            pl.BlockSpec((1, RT, D), lambda b, i: (b, i, 0)),
            pl.BlockSpec((1, RT, 8), lambda b, i: (b, i, 0)),
            pl.BlockSpec((1, 8, N), lambda b, i: (b, 0, 0)),
            pl.BlockSpec((D, D), lambda b, i: (0, 0)),
            pl.BlockSpec((D, D), lambda b, i: (0, 0)),
            pl.BlockSpec((D, D), lambda b, i: (0, 0)),
            pl.BlockSpec((1, D), lambda b, i: (0, 0)),
            pl.BlockSpec((1, D), lambda b, i: (0, 0)),
            pl.BlockSpec((1, D), lambda b, i: (0, 0)),
            pl.BlockSpec((D // 2, D), lambda b, i: (0, 0)),
            pl.BlockSpec((1, D), lambda b, i: (0, 0)),
        ],
        out_specs=[
            pl.BlockSpec((1, RT, D), lambda b, i: (b, i, 0)),
            pl.BlockSpec((1, RT, 2 * D), lambda b, i: (b, i, 0)),
            pl.BlockSpec((D // 2, D), lambda b, i: (0, 0)),
            pl.BlockSpec((D // 2, D), lambda b, i: (0, 0)),
            pl.BlockSpec((1, 2 * D), lambda b, i: (0, 0)),
            pl.BlockSpec((1, RT, K), lambda b, i: (b, i, 0)),
            pl.BlockSpec((1, RT, K), lambda b, i: (b, i, 0)),
        ],
        out_shape=[
            jax.ShapeDtypeStruct((B, N, D), f32),
            jax.ShapeDtypeStruct((B, N, 2 * D), jnp.bfloat16),
            jax.ShapeDtypeStruct((D // 2, D), jnp.bfloat16),
            jax.ShapeDtypeStruct((D // 2, D), jnp.bfloat16),
            jax.ShapeDtypeStruct((1, 2 * D), f32),
            jax.ShapeDtypeStruct((B, N, K), jnp.int32),
            jax.ShapeDtypeStruct((B, N, K), f32),
        ],
    )(fb, pts8, ptsT8, wqT, wkT, wvT, bq, bk, bv, w3T, b3)


def _sc_gather(table, idx_flat):
    nch = _IDX_PER_W // _CH

    @functools.partial(
        pl.kernel,
        mesh=plsc.VectorSubcoreMesh(core_axis_name="c", subcore_axis_name="s"),
        out_type=jax.ShapeDtypeStruct((_TOTAL_PAIRS, D), jnp.int32),
        scratch_types=[
            pltpu.VMEM((_IDX_PER_W,), jnp.int32),
            pltpu.VMEM((_CH, D), jnp.int32),
            pltpu.VMEM((_CH, D), jnp.int32),
            pltpu.SemaphoreType.DMA,
            pltpu.SemaphoreType.DMA,
        ],
    )
    def _gather(table_hbm, idx_hbm, out_hbm, idx_v, rows0, rows1, sem0, sem1):
        wid = lax.axis_index("s") * _NC + lax.axis_index("c")
        base = wid * _IDX_PER_W
        pltpu.sync_copy(idx_hbm.at[pl.ds(base, _IDX_PER_W)], idx_v)
        bufs = (rows0, rows1)
        sems = (sem0, sem1)

        def gath(i):
            return pltpu.async_copy(
                table_hbm.at[idx_v.at[pl.ds(i * _CH, _CH)]],
                bufs[i % 2], sems[i % 2])

        pend = gath(0)
        for i in range(nch):
            nxt = gath(i + 1) if i + 1 < nch else None
            pend.wait()
            pltpu.sync_copy(bufs[i % 2], out_hbm.at[pl.ds(base + i * _CH, _CH)])
            pend = nxt

    return _gather(table, idx_flat)


def _k3_body(kv_ref, nd_ref, qf_ref, fb_ref, akT_ref, avT_ref, ckv_ref,
             w1r_ref, b1r_ref, w2T_ref, b2r_ref,
             owT_ref, ob_ref, w1aT_ref, w1bT_ref, sb1_ref,
             lng_ref, lnb_ref, w2sT_ref, sb2_ref, out_ref):
    f32 = jnp.float32
    kv = kv_ref[0]
    kfg = kv[:, :D].astype(f32)
    vfg = kv[:, D:].astype(f32)
    ndt = nd_ref[0]
    qt = qf_ref[0]
    fbt = fb_ref[0]

    rowp = lax.broadcasted_iota(jnp.int32, (PAIRS, P), 0) // K
    colp = lax.broadcasted_iota(jnp.int32, (PAIRS, P), 1)
    E = (rowp == colp).astype(f32)
    rn = lax.broadcasted_iota(jnp.int32, (PAIRS, K), 0) % K
    cn = lax.broadcasted_iota(jnp.int32, (PAIRS, K), 1)
    Dmask = rn == cn
    hd = lax.broadcasted_iota(jnp.int32, (D, H), 0) // DH
    hh = lax.broadcasted_iota(jnp.int32, (D, H), 1)
    Gs = jnp.where(hd == hh, f32(SCALE), f32(0.0))
    GT = jnp.where(hd == hh, f32(1.0), f32(0.0)).T

    ndsel = jnp.dot(E, ndt, preferred_element_type=f32)
    X = jnp.where(Dmask, ndsel, 0.0)
    Pmat = jnp.broadcast_to(w1r_ref[...], (K, K))
    h1 = jnp.maximum(jnp.dot(X, Pmat, preferred_element_type=f32) + b1r_ref[...], 0.0)
    h2 = jnp.maximum(jnp.dot(h1, w2T_ref[...], preferred_element_type=f32) + b2r_ref[...], 0.0)
    h2b = h2.astype(jnp.bfloat16)

    ckv = ckv_ref[...]
    Kmat = kfg + jnp.dot(h2b, akT_ref[...], preferred_element_type=f32) + ckv[:, :D]
    Vmat = vfg + jnp.dot(h2b, avT_ref[...], preferred_element_type=f32) + ckv[:, D:]

    Qe = jnp.dot(E, qt, preferred_element_type=f32)
    logits = jnp.dot(Qe * Kmat, Gs, preferred_element_type=f32)
    expl = jnp.exp(logits)
    denom = jnp.dot(E.T, expl, preferred_element_type=f32)
    ae = jnp.dot(expl, GT, preferred_element_type=f32)
    o_raw = jnp.dot(E.T, ae * Vmat, preferred_element_type=f32)
    dexp = jnp.dot(denom, GT, preferred_element_type=f32)
    o = o_raw / dexp

    o2 = jnp.dot(o, owT_ref[...], preferred_element_type=f32) + ob_ref[...]
    h2c = (jnp.dot(fbt, w1aT_ref[...], preferred_element_type=f32)
           + jnp.dot(o2, w1bT_ref[...], preferred_element_type=f32)
           + sb1_ref[...])
    mu = jnp.mean(h2c, axis=1, keepdims=True)
    xc = h2c - mu
    var = jnp.mean(xc * xc, axis=1, keepdims=True)
    hn = xc * lax.rsqrt(var + 1e-5) * lng_ref[...] + lnb_ref[...]
    hn = jnp.maximum(hn, 0.0)
    out_ref[0] = jnp.dot(hn, w2sT_ref[...], preferred_element_type=f32) + sb2_ref[...]


def _stage3(kvg, nd, qf, fb, akT, avT, ckv, w1r, b1r, w2T, b2r,
            owT, ob, w1aT, w1bT, sb1, lng, lnb, w2sT, sb2):
    f32 = jnp.float32
    full = lambda shp: pl.BlockSpec(shp, lambda t: tuple(0 for _ in shp))
    return pl.pallas_call(
        _k3_body,
        grid=(B * NT3,),
        in_specs=[
            pl.BlockSpec((1, PAIRS, 2 * D), lambda t: (t, 0, 0)),
            pl.BlockSpec((1, P, K), lambda t: (t, 0, 0)),
            pl.BlockSpec((1, P, D), lambda t: (t, 0, 0)),
            pl.BlockSpec((1, P, D), lambda t: (t, 0, 0)),
            full((D // 2, D)),
            full((D // 2, D)),
            full((1, 2 * D)),
            full((1, K)),
            full((1, K)),
            full((K, D // 2)),
            full((1, D // 2)),
            full((D, D)),
            full((1, D)),
            full((D, D)),
            full((D, D)),
            full((1, D)),
            full((1, D)),
            full((1, D)),
            full((D, D)),
            full((1, D)),
        ],
        out_specs=pl.BlockSpec((1, P, D), lambda t: (t, 0, 0)),
        out_shape=jax.ShapeDtypeStruct((B * NT3, P, D), f32),
    )(kvg.reshape(B * NT3, PAIRS, 2 * D), nd.reshape(B * NT3, P, K),
      qf.reshape(B * NT3, P, D), fb.reshape(B * NT3, P, D),
      akT, avT, ckv, w1r, b1r, w2T, b2r, owT, ob, w1aT, w1bT, sb1,
      lng, lnb, w2sT, sb2)


def kernel(features, points_xyz, de_w1, de_b1, de_w2, de_b2, de_w3, de_b3,
           in_proj_w, in_proj_b, out_proj_w, out_proj_b,
           se_w1, se_b1, ln_g, ln_b, se_w2, se_b2):
    f32 = jnp.float32
    pts8 = jnp.concatenate(
        [points_xyz, jnp.zeros((B, N, 5), f32)], axis=2)
    ptsT8 = pts8.transpose(0, 2, 1)

    wqT = in_proj_w[:D].T
    wkT = in_proj_w[D:2 * D].T
    wvT = in_proj_w[2 * D:].T
    bq = in_proj_b[:D][None]
    bk = in_proj_b[D:2 * D][None]
    bv = in_proj_b[2 * D:][None]

    qf, kvt, akT, avT, ckv, idx, nd = _stage1(
        features, pts8, ptsT8, wqT, wkT, wvT, bq, bk, bv,
        de_w3.T, de_b3[None])

    table = lax.bitcast_convert_type(
        kvt.reshape(B * N, D, 2), jnp.int32)
    kvg_i = _sc_gather(table, idx.reshape(-1))
    kvg = lax.bitcast_convert_type(
        kvg_i, jnp.bfloat16).reshape(_TOTAL_PAIRS, 2 * D)

    out = _stage3(
        kvg, nd, qf, features, akT, avT, ckv,
        de_w1.T, de_b1[None], de_w2.T, de_b2[None],
        out_proj_w.T, out_proj_b[None],
        se_w1[:, :D].T, se_w1[:, D:].T, se_b1[None],
        ln_g[None], ln_b[None], se_w2.T, se_b2[None])
    return out.reshape(B, N, D)


# speedup vs baseline: 2.4761x; 2.4761x over previous
import functools

import jax
import jax.numpy as jnp
import numpy as np
from jax import lax
from jax.experimental import pallas as pl
from jax.experimental.pallas import tpu as pltpu
from jax.experimental.pallas import tpu_sc as plsc

B, N, D = 2, 2048, 256
K = 64
H = 8
DH = D // H
SCALE = 1.0 / float(np.sqrt(DH))
RT = 256
P = 32
PAIRS = P * K
NT1 = N // RT
NT3 = N // P

_NC, _NS = 2, 16
_NW = _NC * _NS
_TOTAL_PAIRS = B * N * K
_IDX_PER_W = _TOTAL_PAIRS // _NW
_CH = 128


def _k1_body(fb_ref, pts8_ref, ptsT8_ref, wqT_ref, wkT_ref, wvT_ref,
             bq_ref, bk_ref, bv_ref, w3T_ref, b3_ref,
             qf_ref, kv_ref, akT_ref, avT_ref, ckv_ref,
             idx_ref, nd_ref):
    b = pl.program_id(0)
    fb = fb_ref[0]
    wkT = wkT_ref[...]
    wvT = wvT_ref[...]
    qf_ref[0] = jnp.dot(fb, wqT_ref[...], preferred_element_type=jnp.float32) + bq_ref[...]
    kf = jnp.dot(fb, wkT, preferred_element_type=jnp.float32) + bk_ref[...]
    vf = jnp.dot(fb, wvT, preferred_element_type=jnp.float32) + bv_ref[...]
    k16 = lax.bitcast_convert_type(kf.astype(jnp.bfloat16), jnp.uint16)
    v16 = lax.bitcast_convert_type(vf.astype(jnp.bfloat16), jnp.uint16)
    packed = k16.astype(jnp.uint32) | (v16.astype(jnp.uint32) << 16)
    kv_ref[0] = lax.bitcast_convert_type(packed, jnp.int32)
    w3T = w3T_ref[...]
    akT_ref[...] = jnp.dot(w3T, wkT, preferred_element_type=jnp.float32).astype(jnp.bfloat16)
    avT_ref[...] = jnp.dot(w3T, wvT, preferred_element_type=jnp.float32).astype(jnp.bfloat16)
    b3 = b3_ref[...]
    ck = jnp.dot(b3, wkT, preferred_element_type=jnp.float32)
    cv = jnp.dot(b3, wvT, preferred_element_type=jnp.float32)
    ckv_ref[...] = jnp.concatenate([ck, cv], axis=1)

    pts = pts8_ref[0]
    ptsT = ptsT8_ref[0]
    sq_row = jnp.sum(pts * pts, axis=1, keepdims=True)
    sq_all = jnp.sum(ptsT * ptsT, axis=0, keepdims=True)
    g = jnp.dot(pts, ptsT, preferred_element_type=jnp.float32)
    d2 = jnp.maximum(sq_row + sq_all - 2.0 * g, 1e-12)

    iota = lax.broadcasted_iota(jnp.int32, (RT, N), 1)
    slot = lax.broadcasted_iota(jnp.int32, (1, K), 1)

    def body(t, carry):
        work, acc_i, acc_n = carry
        m = jnp.min(work, axis=1, keepdims=True)
        sel = jnp.where(work == m, iota, jnp.int32(1 << 30))
        j = jnp.min(sel, axis=1, keepdims=True)
        onehot = slot == t
        acc_i = acc_i + jnp.where(onehot, j, 0)
        acc_n = acc_n + jnp.where(onehot, m, 0.0)
        work = jnp.where(iota == j, jnp.float32(np.inf), work)
        return work, acc_i, acc_n

    _, acc_i, acc_n = lax.fori_loop(
        0, K, body,
        (d2, jnp.zeros((RT, K), jnp.int32), jnp.zeros((RT, K), jnp.float32)))
    idx_ref[0] = acc_i + b * N
    nd_ref[0] = jnp.sqrt(acc_n)


def _stage1(fb, pts8, ptsT8, wqT, wkT, wvT, bq, bk, bv, w3T, b3):
    f32 = jnp.float32
    return pl.pallas_call(
        _k1_body,
        grid=(B, NT1),
        in_specs=[
            pl.BlockSpec((1, RT, D), lambda b, i: (b, i, 0)),
            pl.BlockSpec((1, RT, 8), lambda b, i: (b, i, 0)),
            pl.BlockSpec((1, 8, N), lambda b, i: (b, 0, 0)),
            pl.BlockSpec((D, D), lambda b, i: (0, 0)),
            pl.BlockSpec((D, D), lambda b, i: (0, 0)),
            pl.BlockSpec((D, D), lambda b, i: (0, 0)),
            pl.BlockSpec((1, D), lambda b, i: (0, 0)),
            pl.BlockSpec((1, D), lambda b, i: (0, 0)),
            pl.BlockSpec((1, D), lambda b, i: (0, 0)),
            pl.BlockSpec((D // 2, D), lambda b, i: (0, 0)),
            pl.BlockSpec((1, D), lambda b, i: (0, 0)),
        ],
        out_specs=[
            pl.BlockSpec((1, RT, D), lambda b, i: (b, i, 0)),
            pl.BlockSpec((1, RT, D), lambda b, i: (b, i, 0)),
            pl.BlockSpec((D // 2, D), lambda b, i: (0, 0)),
            pl.BlockSpec((D // 2, D), lambda b, i: (0, 0)),
            pl.BlockSpec((1, 2 * D), lambda b, i: (0, 0)),
            pl.BlockSpec((1, RT, K), lambda b, i: (b, i, 0)),
            pl.BlockSpec((1, RT, K), lambda b, i: (b, i, 0)),
        ],
        out_shape=[
            jax.ShapeDtypeStruct((B, N, D), f32),
            jax.ShapeDtypeStruct((B, N, D), jnp.int32),
            jax.ShapeDtypeStruct((D // 2, D), jnp.bfloat16),
            jax.ShapeDtypeStruct((D // 2, D), jnp.bfloat16),
            jax.ShapeDtypeStruct((1, 2 * D), f32),
            jax.ShapeDtypeStruct((B, N, K), jnp.int32),
            jax.ShapeDtypeStruct((B, N, K), f32),
        ],
    )(fb, pts8, ptsT8, wqT, wkT, wvT, bq, bk, bv, w3T, b3)


def _sc_gather(table, idx_flat):
    nch = _IDX_PER_W // _CH

    @functools.partial(
        pl.kernel,
        mesh=plsc.VectorSubcoreMesh(core_axis_name="c", subcore_axis_name="s"),
        out_type=jax.ShapeDtypeStruct((_TOTAL_PAIRS, D), jnp.int32),
        scratch_types=[
            pltpu.VMEM((_IDX_PER_W,), jnp.int32),
            pltpu.VMEM((_CH, D), jnp.int32),
            pltpu.VMEM((_CH, D), jnp.int32),
            pltpu.SemaphoreType.DMA,
            pltpu.SemaphoreType.DMA,
        ],
    )
    def _gather(table_hbm, idx_hbm, out_hbm, idx_v, rows0, rows1, sem0, sem1):
        wid = lax.axis_index("s") * _NC + lax.axis_index("c")
        base = wid * _IDX_PER_W
        pltpu.sync_copy(idx_hbm.at[pl.ds(base, _IDX_PER_W)], idx_v)
        bufs = (rows0, rows1)
        sems = (sem0, sem1)

        def gath(i):
            return pltpu.async_copy(
                table_hbm.at[idx_v.at[pl.ds(i * _CH, _CH)]],
                bufs[i % 2], sems[i % 2])

        pend = gath(0)
        for i in range(nch):
            nxt = gath(i + 1) if i + 1 < nch else None
            pend.wait()
            pltpu.sync_copy(bufs[i % 2], out_hbm.at[pl.ds(base + i * _CH, _CH)])
            pend = nxt

    return _gather(table, idx_flat)


def _k3_body(kv_ref, nd_ref, qf_ref, fb_ref, akT_ref, avT_ref, ckv_ref,
             w1r_ref, b1r_ref, w2T_ref, b2r_ref,
             owT_ref, ob_ref, w1aT_ref, w1bT_ref, sb1_ref,
             lng_ref, lnb_ref, w2sT_ref, sb2_ref, out_ref):
    f32 = jnp.float32
    u = lax.bitcast_convert_type(kv_ref[0], jnp.uint32)
    kfg = lax.bitcast_convert_type(
        (u & 0xFFFF).astype(jnp.uint16), jnp.bfloat16).astype(f32)
    vfg = lax.bitcast_convert_type(
        (u >> 16).astype(jnp.uint16), jnp.bfloat16).astype(f32)
    ndt = nd_ref[0]
    qt = qf_ref[0]
    fbt = fb_ref[0]

    rowp = lax.broadcasted_iota(jnp.int32, (PAIRS, P), 0) // K
    colp = lax.broadcasted_iota(jnp.int32, (PAIRS, P), 1)
    E = (rowp == colp).astype(f32)
    rn = lax.broadcasted_iota(jnp.int32, (PAIRS, K), 0) % K
    cn = lax.broadcasted_iota(jnp.int32, (PAIRS, K), 1)
    Dmask = rn == cn
    hd = lax.broadcasted_iota(jnp.int32, (D, H), 0) // DH
    hh = lax.broadcasted_iota(jnp.int32, (D, H), 1)
    Gs = jnp.where(hd == hh, f32(SCALE), f32(0.0))
    GT = jnp.where(hd == hh, f32(1.0), f32(0.0)).T

    ndsel = jnp.dot(E, ndt, preferred_element_type=f32)
    X = jnp.where(Dmask, ndsel, 0.0)
    Pmat = jnp.broadcast_to(w1r_ref[...], (K, K))
    h1 = jnp.maximum(jnp.dot(X, Pmat, preferred_element_type=f32) + b1r_ref[...], 0.0)
    h2 = jnp.maximum(jnp.dot(h1, w2T_ref[...], preferred_element_type=f32) + b2r_ref[...], 0.0)
    h2b = h2.astype(jnp.bfloat16)

    ckv = ckv_ref[...]
    Kmat = kfg + jnp.dot(h2b, akT_ref[...], preferred_element_type=f32) + ckv[:, :D]
    Vmat = vfg + jnp.dot(h2b, avT_ref[...], preferred_element_type=f32) + ckv[:, D:]

    Qe = jnp.dot(E, qt, preferred_element_type=f32)
    logits = jnp.dot(Qe * Kmat, Gs, preferred_element_type=f32)
    expl = jnp.exp(logits)
    denom = jnp.dot(E.T, expl, preferred_element_type=f32)
    ae = jnp.dot(expl, GT, preferred_element_type=f32)
    o_raw = jnp.dot(E.T, ae * Vmat, preferred_element_type=f32)
    dexp = jnp.dot(denom, GT, preferred_element_type=f32)
    o = o_raw / dexp

    o2 = jnp.dot(o, owT_ref[...], preferred_element_type=f32) + ob_ref[...]
    h2c = (jnp.dot(fbt, w1aT_ref[...], preferred_element_type=f32)
           + jnp.dot(o2, w1bT_ref[...], preferred_element_type=f32)
           + sb1_ref[...])
    mu = jnp.mean(h2c, axis=1, keepdims=True)
    xc = h2c - mu
    var = jnp.mean(xc * xc, axis=1, keepdims=True)
    hn = xc * lax.rsqrt(var + 1e-5) * lng_ref[...] + lnb_ref[...]
    hn = jnp.maximum(hn, 0.0)
    out_ref[0] = jnp.dot(hn, w2sT_ref[...], preferred_element_type=f32) + sb2_ref[...]


def _stage3(kvg, nd, qf, fb, akT, avT, ckv, w1r, b1r, w2T, b2r,
            owT, ob, w1aT, w1bT, sb1, lng, lnb, w2sT, sb2):
    f32 = jnp.float32
    full = lambda shp: pl.BlockSpec(shp, lambda t: tuple(0 for _ in shp))
    return pl.pallas_call(
        _k3_body,
        grid=(B * NT3,),
        in_specs=[
            pl.BlockSpec((1, PAIRS, D), lambda t: (t, 0, 0)),
            pl.BlockSpec((1, P, K), lambda t: (t, 0, 0)),
            pl.BlockSpec((1, P, D), lambda t: (t, 0, 0)),
            pl.BlockSpec((1, P, D), lambda t: (t, 0, 0)),
            full((D // 2, D)),
            full((D // 2, D)),
            full((1, 2 * D)),
            full((1, K)),
            full((1, K)),
            full((K, D // 2)),
            full((1, D // 2)),
            full((D, D)),
            full((1, D)),
            full((D, D)),
            full((D, D)),
            full((1, D)),
            full((1, D)),
            full((1, D)),
            full((D, D)),
            full((1, D)),
        ],
        out_specs=pl.BlockSpec((1, P, D), lambda t: (t, 0, 0)),
        out_shape=jax.ShapeDtypeStruct((B * NT3, P, D), f32),
    )(kvg.reshape(B * NT3, PAIRS, D), nd.reshape(B * NT3, P, K),
      qf.reshape(B * NT3, P, D), fb.reshape(B * NT3, P, D),
      akT, avT, ckv, w1r, b1r, w2T, b2r, owT, ob, w1aT, w1bT, sb1,
      lng, lnb, w2sT, sb2)


def kernel(features, points_xyz, de_w1, de_b1, de_w2, de_b2, de_w3, de_b3,
           in_proj_w, in_proj_b, out_proj_w, out_proj_b,
           se_w1, se_b1, ln_g, ln_b, se_w2, se_b2):
    f32 = jnp.float32
    pts8 = jnp.concatenate(
        [points_xyz, jnp.zeros((B, N, 5), f32)], axis=2)
    ptsT8 = pts8.transpose(0, 2, 1)

    wqT = in_proj_w[:D].T
    wkT = in_proj_w[D:2 * D].T
    wvT = in_proj_w[2 * D:].T
    bq = in_proj_b[:D][None]
    bk = in_proj_b[D:2 * D][None]
    bv = in_proj_b[2 * D:][None]

    qf, kvt, akT, avT, ckv, idx, nd = _stage1(
        features, pts8, ptsT8, wqT, wkT, wvT, bq, bk, bv,
        de_w3.T, de_b3[None])

    table = kvt.reshape(B * N, D)
    kvg = _sc_gather(table, idx.reshape(-1))

    out = _stage3(
        kvg, nd, qf, features, akT, avT, ckv,
        de_w1.T, de_b1[None], de_w2.T, de_b2[None],
        out_proj_w.T, out_proj_b[None],
        se_w1[:, :D].T, se_w1[:, D:].T, se_b1[None],
        ln_g[None], ln_b[None], se_w2.T, se_b2[None])
    return out.reshape(B, N, D)
